# Initial kernel scaffold; baseline (speedup 1.0000x reference)
import jax
import jax.numpy as jnp
from jax.experimental import pallas as pl


def kernel(x, edge_index, W1, b1, W2, b2, W3, b3):
    raise NotImplementedError("write your pallas kernel here")


# speedup vs baseline: 7.8543x; 7.8543x over previous
import functools

import jax
import jax.numpy as jnp
from jax import lax
from jax.experimental import pallas as pl
from jax.experimental.pallas import tpu as pltpu
from jax.experimental.pallas import tpu_sc as plsc

N_NODES = 10000
N_EDGES = 320000
CH = 128
ECH = 128
NC = 2
NS = 16
NW = NC * NS
CPW = -(-N_EDGES // (NW * ECH))
E_PAD = NW * ECH * CPW
N_PAD = 10112
RPT = N_PAD // NS

_MESH = dict(core_axis_name="c", subcore_axis_name="s", num_cores=NC,
             num_subcores=NS)


def _zero_fill(buf, rows, width):
    z = jnp.zeros((16,), jnp.float32)

    @pl.loop(0, rows)
    def _(r):
        for c in range(width // 16):
            buf[r, pl.ds(c * 16, 16)] = z


@functools.cache
def _make_agg_sc():
    return functools.partial(
        pl.kernel,
        out_type=jax.ShapeDtypeStruct((NC, N_PAD, CH), jnp.float32),
        mesh=plsc.VectorSubcoreMesh(**_MESH),
        scratch_types=[
            pltpu.VMEM((ECH,), jnp.int32),
            pltpu.VMEM((ECH,), jnp.int32),
            pltpu.VMEM((ECH, CH), jnp.float32),
            pltpu.VMEM_SHARED((N_PAD, CH), jnp.float32),
            pltpu.SemaphoreType.DMA,
            pltpu.SemaphoreType.DMA,
        ],
    )(_agg_body)


def _agg_body(g_hbm, src_hbm, dst_hbm, out_hbm, srcc_v, dstc_v, rows_v,
              acc_sh, sem0, sem1):
    cid = lax.axis_index("c")
    sid = lax.axis_index("s")
    wid = cid * NS + sid

    _zero_fill(rows_v, ECH, CH)
    for k in range(RPT // ECH):
        pltpu.sync_copy(rows_v, acc_sh.at[pl.ds(sid * RPT + k * ECH, ECH)])
    rem = RPT % ECH
    if rem:
        pltpu.sync_copy(rows_v.at[pl.ds(0, rem)],
                        acc_sh.at[pl.ds(sid * RPT + (RPT // ECH) * ECH, rem)])
    plsc.subcore_barrier()

    @pl.loop(0, CPW)
    def _(j):
        cs = pltpu.async_copy(src_hbm.at[wid, j], srcc_v, sem0)
        cd = pltpu.async_copy(dst_hbm.at[wid, j], dstc_v, sem1)
        cs.wait()
        cd.wait()
        pltpu.async_copy(g_hbm.at[srcc_v], rows_v, sem0).wait()
        pltpu.sync_copy(rows_v, acc_sh.at[dstc_v], add=True)

    plsc.subcore_barrier()

    base = sid * RPT
    for k in range(RPT // ECH):
        pltpu.sync_copy(acc_sh.at[pl.ds(base + k * ECH, ECH)], rows_v)
        pltpu.sync_copy(rows_v, out_hbm.at[cid, pl.ds(base + k * ECH, ECH)])
    if rem:
        b2 = base + (RPT // ECH) * ECH
        pltpu.sync_copy(acc_sh.at[pl.ds(b2, rem)], rows_v.at[pl.ds(0, rem)])
        pltpu.sync_copy(rows_v.at[pl.ds(0, rem)],
                        out_hbm.at[cid, pl.ds(b2, rem)])


_BLK = 1000
_GRID = N_NODES // _BLK


def _prep_body(x_ref, w_ref, degp_ref, g_ref, dis_ref):
    deg = degp_ref[0, :, 0:1] + degp_ref[1, :, 0:1] + 1.0
    d = lax.rsqrt(deg)
    dis_ref[...] = d
    g_ref[...] = d * jnp.dot(x_ref[...], w_ref[...],
                             preferred_element_type=jnp.float32)


_prep_tc = pl.pallas_call(
    _prep_body,
    grid=(_GRID,),
    in_specs=[
        pl.BlockSpec((_BLK, CH), lambda i: (i, 0)),
        pl.BlockSpec((CH, CH), lambda i: (0, 0)),
        pl.BlockSpec((NC, _BLK, CH), lambda i: (0, i, 0)),
    ],
    out_specs=[
        pl.BlockSpec((_BLK, CH), lambda i: (i, 0)),
        pl.BlockSpec((_BLK, 1), lambda i: (i, 0)),
    ],
    out_shape=[
        jax.ShapeDtypeStruct((N_NODES, CH), jnp.float32),
        jax.ShapeDtypeStruct((N_NODES, 1), jnp.float32),
    ],
)


def _comb_body(p_ref, g_ref, dis_ref, b_ref, w_ref, gn_ref):
    d = dis_ref[...]
    h = d * (p_ref[0] + p_ref[1] + g_ref[...]) + b_ref[...]
    r = jnp.maximum(h, 0.0)
    gn_ref[...] = d * jnp.dot(r, w_ref[...],
                              preferred_element_type=jnp.float32)


_comb_tc = pl.pallas_call(
    _comb_body,
    grid=(_GRID,),
    in_specs=[
        pl.BlockSpec((NC, _BLK, CH), lambda i: (0, i, 0)),
        pl.BlockSpec((_BLK, CH), lambda i: (i, 0)),
        pl.BlockSpec((_BLK, 1), lambda i: (i, 0)),
        pl.BlockSpec((1, CH), lambda i: (0, 0)),
        pl.BlockSpec((CH, CH), lambda i: (0, 0)),
    ],
    out_specs=pl.BlockSpec((_BLK, CH), lambda i: (i, 0)),
    out_shape=jax.ShapeDtypeStruct((N_NODES, CH), jnp.float32),
)


def _final_body(p_ref, g_ref, dis_ref, b_ref, o_ref):
    o_ref[...] = (dis_ref[...] * (p_ref[0] + p_ref[1] + g_ref[...])
                  + b_ref[...])


_final_tc = pl.pallas_call(
    _final_body,
    grid=(_GRID,),
    in_specs=[
        pl.BlockSpec((NC, _BLK, CH), lambda i: (0, i, 0)),
        pl.BlockSpec((_BLK, CH), lambda i: (i, 0)),
        pl.BlockSpec((_BLK, 1), lambda i: (i, 0)),
        pl.BlockSpec((1, CH), lambda i: (0, 0)),
    ],
    out_specs=pl.BlockSpec((_BLK, CH), lambda i: (i, 0)),
    out_shape=jax.ShapeDtypeStruct((N_NODES, CH), jnp.float32),
)


@jax.jit
def kernel(x, edge_index, W1, b1, W2, b2, W3, b3):
    pad = E_PAD - N_EDGES
    src = jnp.concatenate(
        [edge_index[0], jnp.zeros((pad,), jnp.int32)]).reshape(NW, CPW, ECH)
    dst = jnp.concatenate(
        [edge_index[1],
         jnp.full((pad,), N_NODES, jnp.int32)]).reshape(NW, CPW, ECH)

    agg_sc = _make_agg_sc()
    ones = jnp.ones((N_NODES, CH), jnp.float32)
    degp = agg_sc(ones, src, dst)
    g1, dis = _prep_tc(x, W1, degp)
    p1 = agg_sc(g1, src, dst)
    g2 = _comb_tc(p1, g1, dis, b1.reshape(1, CH), W2)
    p2 = agg_sc(g2, src, dst)
    g3 = _comb_tc(p2, g2, dis, b2.reshape(1, CH), W3)
    p3 = agg_sc(g3, src, dst)
    return _final_tc(p3, g3, dis, b3.reshape(1, CH))


# speedup vs baseline: 7.8625x; 1.0010x over previous
import functools

import jax
import jax.numpy as jnp
from jax import lax
from jax.experimental import pallas as pl
from jax.experimental.pallas import tpu as pltpu
from jax.experimental.pallas import tpu_sc as plsc

N_NODES = 10000
N_EDGES = 320000
CH = 128
ECH = 128
NC = 2
NS = 16
NW = NC * NS
CPW = 80
CPW_ALLOC = CPW + 2
N_PAD = 10112
RPT = N_PAD // NS
_GROUPS = 4
_GLEN = CPW // _GROUPS

_MESH = dict(core_axis_name="c", subcore_axis_name="s", num_cores=NC,
             num_subcores=NS)

_AGG_SCRATCH = [
    pltpu.VMEM((ECH,), jnp.int32),
    pltpu.VMEM((ECH,), jnp.int32),
    pltpu.VMEM((ECH,), jnp.int32),
    pltpu.VMEM((ECH,), jnp.int32),
    pltpu.VMEM((ECH, CH), jnp.float32),
    pltpu.VMEM((ECH, CH), jnp.float32),
    pltpu.VMEM_SHARED((N_PAD, CH), jnp.float32),
    pltpu.SemaphoreType.DMA,
    pltpu.SemaphoreType.DMA,
    pltpu.SemaphoreType.DMA,
    pltpu.SemaphoreType.DMA,
    pltpu.SemaphoreType.DMA,
    pltpu.SemaphoreType.DMA,
]


def _zero_fill(buf, rows, width):
    z = jnp.zeros((16,), jnp.float32)

    @pl.loop(0, rows)
    def _(r):
        for c in range(width // 16):
            buf[r, pl.ds(c * 16, 16)] = z


def _zero_acc(zbuf, acc_sh, sid):
    _zero_fill(zbuf, ECH, CH)
    for k in range(RPT // ECH):
        pltpu.sync_copy(zbuf, acc_sh.at[pl.ds(sid * RPT + k * ECH, ECH)])
    rem = RPT % ECH
    if rem:
        pltpu.sync_copy(zbuf.at[pl.ds(0, rem)],
                        acc_sh.at[pl.ds(sid * RPT + (RPT // ECH) * ECH, rem)])


def _copy_out(acc_sh, rows0, rows1, out_hbm, cid, sid):
    base = sid * RPT
    rem = RPT % ECH
    bufs = (rows0, rows1)
    for k in range(RPT // ECH):
        b = bufs[k % 2]
        pltpu.sync_copy(acc_sh.at[pl.ds(base + k * ECH, ECH)], b)
        pltpu.sync_copy(b, out_hbm.at[cid, pl.ds(base + k * ECH, ECH)])
    if rem:
        b2 = base + (RPT // ECH) * ECH
        pltpu.sync_copy(acc_sh.at[pl.ds(b2, rem)], rows0.at[pl.ds(0, rem)])
        pltpu.sync_copy(rows0.at[pl.ds(0, rem)],
                        out_hbm.at[cid, pl.ds(b2, rem)])


@functools.cache
def _make_agg_sc(with_gather: bool):
    body = _agg_body if with_gather else _deg_body
    return functools.partial(
        pl.kernel,
        out_type=jax.ShapeDtypeStruct((NC, N_PAD, CH), jnp.float32),
        mesh=plsc.VectorSubcoreMesh(**_MESH),
        scratch_types=_AGG_SCRATCH,
    )(body)


def _agg_body(g_hbm, src_hbm, dst_hbm, out_hbm, srcc0, srcc1, dstc0, dstc1,
              rows0, rows1, acc_sh, ss0, ss1, sd0, sd1, sg0, sg1):
    cid = lax.axis_index("c")
    sid = lax.axis_index("s")
    wid = cid * NS + sid
    srcc = (srcc0, srcc1)
    dstc = (dstc0, dstc1)
    rows = (rows0, rows1)
    sems = (ss0, ss1)
    semd = (sd0, sd1)
    semg = (sg0, sg1)

    _zero_acc(rows0, acc_sh, sid)
    plsc.subcore_barrier()

    def fetch_s(j, par):
        return pltpu.async_copy(src_hbm.at[wid, j], srcc[par], sems[par])

    def fetch_d(j, par):
        return pltpu.async_copy(dst_hbm.at[wid, j], dstc[par], semd[par])

    def gather(par):
        return pltpu.async_copy(g_hbm.at[srcc[par]], rows[par], semg[par])

    @pl.loop(0, _GROUPS)
    def _(p):
        j0 = p * _GLEN
        fs = {0: fetch_s(j0, 0), 1: fetch_s(j0 + 1, 1)}
        fd = {0: fetch_d(j0, 0), 1: fetch_d(j0 + 1, 1)}
        fs.pop(0).wait()
        g = {0: gather(0)}
        for t in range(_GLEN):
            par = t % 2
            g.pop(t).wait()
            if t + 1 < _GLEN:
                fs.pop(t + 1).wait()
                g[t + 1] = gather(1 - par)
            fd.pop(t).wait()
            pltpu.sync_copy(rows[par], acc_sh.at[dstc[par]], add=True)
            if t + 2 < _GLEN:
                fs[t + 2] = fetch_s(j0 + t + 2, par)
                fd[t + 2] = fetch_d(j0 + t + 2, par)

    plsc.subcore_barrier()
    _copy_out(acc_sh, rows0, rows1, out_hbm, cid, sid)


def _deg_body(g_hbm, src_hbm, dst_hbm, out_hbm, srcc0, srcc1, dstc0, dstc1,
              rows0, rows1, acc_sh, ss0, ss1, sd0, sd1, sg0, sg1):
    cid = lax.axis_index("c")
    sid = lax.axis_index("s")
    wid = cid * NS + sid
    dstc = (dstc0, dstc1)
    semd = (sd0, sd1)

    _zero_acc(rows0, acc_sh, sid)

    one = jnp.ones((16,), jnp.float32)

    @pl.loop(0, ECH)
    def _(r):
        for c in range(CH // 16):
            rows1[r, pl.ds(c * 16, 16)] = one

    plsc.subcore_barrier()

    def fetch_d(j, par):
        return pltpu.async_copy(dst_hbm.at[wid, j], dstc[par], semd[par])

    @pl.loop(0, _GROUPS)
    def _(p):
        j0 = p * _GLEN
        fd = {0: fetch_d(j0, 0), 1: fetch_d(j0 + 1, 1)}
        for t in range(_GLEN):
            par = t % 2
            fd.pop(t).wait()
            pltpu.sync_copy(rows1, acc_sh.at[dstc[par]], add=True)
            if t + 2 < _GLEN:
                fd[t + 2] = fetch_d(j0 + t + 2, par)

    plsc.subcore_barrier()
    _copy_out(acc_sh, rows0, rows1, out_hbm, cid, sid)


_BLK = 1000
_GRID = N_NODES // _BLK


def _prep_body(x_ref, w_ref, degp_ref, g_ref, dis_ref):
    deg = degp_ref[0, :, 0:1] + degp_ref[1, :, 0:1] + 1.0
    d = lax.rsqrt(deg)
    dis_ref[...] = d
    g_ref[...] = d * jnp.dot(x_ref[...], w_ref[...],
                             preferred_element_type=jnp.float32)


_prep_tc = pl.pallas_call(
    _prep_body,
    grid=(_GRID,),
    in_specs=[
        pl.BlockSpec((_BLK, CH), lambda i: (i, 0)),
        pl.BlockSpec((CH, CH), lambda i: (0, 0)),
        pl.BlockSpec((NC, _BLK, CH), lambda i: (0, i, 0)),
    ],
    out_specs=[
        pl.BlockSpec((_BLK, CH), lambda i: (i, 0)),
        pl.BlockSpec((_BLK, 1), lambda i: (i, 0)),
    ],
    out_shape=[
        jax.ShapeDtypeStruct((N_NODES, CH), jnp.float32),
        jax.ShapeDtypeStruct((N_NODES, 1), jnp.float32),
    ],
)


def _comb_body(p_ref, g_ref, dis_ref, b_ref, w_ref, gn_ref):
    d = dis_ref[...]
    h = d * (p_ref[0] + p_ref[1] + g_ref[...]) + b_ref[...]
    r = jnp.maximum(h, 0.0)
    gn_ref[...] = d * jnp.dot(r, w_ref[...],
                              preferred_element_type=jnp.float32)


_comb_tc = pl.pallas_call(
    _comb_body,
    grid=(_GRID,),
    in_specs=[
        pl.BlockSpec((NC, _BLK, CH), lambda i: (0, i, 0)),
        pl.BlockSpec((_BLK, CH), lambda i: (i, 0)),
        pl.BlockSpec((_BLK, 1), lambda i: (i, 0)),
        pl.BlockSpec((1, CH), lambda i: (0, 0)),
        pl.BlockSpec((CH, CH), lambda i: (0, 0)),
    ],
    out_specs=pl.BlockSpec((_BLK, CH), lambda i: (i, 0)),
    out_shape=jax.ShapeDtypeStruct((N_NODES, CH), jnp.float32),
)


def _final_body(p_ref, g_ref, dis_ref, b_ref, o_ref):
    o_ref[...] = (dis_ref[...] * (p_ref[0] + p_ref[1] + g_ref[...])
                  + b_ref[...])


_final_tc = pl.pallas_call(
    _final_body,
    grid=(_GRID,),
    in_specs=[
        pl.BlockSpec((NC, _BLK, CH), lambda i: (0, i, 0)),
        pl.BlockSpec((_BLK, CH), lambda i: (i, 0)),
        pl.BlockSpec((_BLK, 1), lambda i: (i, 0)),
        pl.BlockSpec((1, CH), lambda i: (0, 0)),
    ],
    out_specs=pl.BlockSpec((_BLK, CH), lambda i: (i, 0)),
    out_shape=jax.ShapeDtypeStruct((N_NODES, CH), jnp.float32),
)


@jax.jit
def kernel(x, edge_index, W1, b1, W2, b2, W3, b3):
    pad = NW * CPW * ECH - N_EDGES
    src = jnp.concatenate(
        [edge_index[0], jnp.zeros((pad,), jnp.int32)]).reshape(NW, CPW, ECH)
    dst = jnp.concatenate(
        [edge_index[1],
         jnp.full((pad,), N_NODES, jnp.int32)]).reshape(NW, CPW, ECH)
    src = jnp.concatenate([src, jnp.zeros((NW, 2, ECH), jnp.int32)], axis=1)
    dst = jnp.concatenate(
        [dst, jnp.full((NW, 2, ECH), N_NODES, jnp.int32)], axis=1)

    agg_sc = _make_agg_sc(True)
    deg_sc = _make_agg_sc(False)
    degp = deg_sc(x, src, dst)
    g1, dis = _prep_tc(x, W1, degp)
    p1 = agg_sc(g1, src, dst)
    g2 = _comb_tc(p1, g1, dis, b1.reshape(1, CH), W2)
    p2 = agg_sc(g2, src, dst)
    g3 = _comb_tc(p2, g2, dis, b2.reshape(1, CH), W3)
    p3 = agg_sc(g3, src, dst)
    return _final_tc(p3, g3, dis, b3.reshape(1, CH))


# speedup vs baseline: 22.4064x; 2.8498x over previous
import functools

import jax
import jax.numpy as jnp
from jax import lax
from jax.experimental import pallas as pl
from jax.experimental.pallas import tpu as pltpu
from jax.experimental.pallas import tpu_sc as plsc

N_NODES = 10000
N_EDGES = 320000
CH = 128
ECH = 128
NC = 2
NS = 16
NW = NC * NS
CPW = 80
CPW_ALLOC = CPW + 2
N_PAD = 10112
RPT = N_PAD // NS
_GROUPS = 4
_GLEN = CPW // _GROUPS

_MESH = dict(core_axis_name="c", subcore_axis_name="s", num_cores=NC,
             num_subcores=NS)

_AGG_SCRATCH = [
    pltpu.VMEM((ECH,), jnp.int32),
    pltpu.VMEM((ECH,), jnp.int32),
    pltpu.VMEM((ECH,), jnp.int32),
    pltpu.VMEM((ECH,), jnp.int32),
    pltpu.VMEM((ECH, CH), jnp.float32),
    pltpu.VMEM((ECH, CH), jnp.float32),
    pltpu.VMEM_SHARED((N_PAD, CH), jnp.float32),
    pltpu.SemaphoreType.DMA,
    pltpu.SemaphoreType.DMA,
    pltpu.SemaphoreType.DMA,
    pltpu.SemaphoreType.DMA,
    pltpu.SemaphoreType.DMA,
    pltpu.SemaphoreType.DMA,
]


def _zero_fill(buf, rows, width):
    z = jnp.zeros((16,), jnp.float32)

    @pl.loop(0, rows)
    def _(r):
        for c in range(width // 16):
            buf[r, pl.ds(c * 16, 16)] = z


def _zero_acc(zbuf, acc_sh, sid):
    _zero_fill(zbuf, ECH, CH)
    for k in range(RPT // ECH):
        pltpu.sync_copy(zbuf, acc_sh.at[pl.ds(sid * RPT + k * ECH, ECH)])
    rem = RPT % ECH
    if rem:
        pltpu.sync_copy(zbuf.at[pl.ds(0, rem)],
                        acc_sh.at[pl.ds(sid * RPT + (RPT // ECH) * ECH, rem)])


def _copy_out(acc_sh, rows0, rows1, out_hbm, cid, sid):
    base = sid * RPT
    rem = RPT % ECH
    bufs = (rows0, rows1)
    for k in range(RPT // ECH):
        b = bufs[k % 2]
        pltpu.sync_copy(acc_sh.at[pl.ds(base + k * ECH, ECH)], b)
        pltpu.sync_copy(b, out_hbm.at[cid, pl.ds(base + k * ECH, ECH)])
    if rem:
        b2 = base + (RPT // ECH) * ECH
        pltpu.sync_copy(acc_sh.at[pl.ds(b2, rem)], rows0.at[pl.ds(0, rem)])
        pltpu.sync_copy(rows0.at[pl.ds(0, rem)],
                        out_hbm.at[cid, pl.ds(b2, rem)])


@functools.cache
def _make_agg_sc(with_gather: bool):
    body = _agg_body if with_gather else _deg_body
    return functools.partial(
        pl.kernel,
        out_type=jax.ShapeDtypeStruct((NC, N_PAD, CH), jnp.float32),
        mesh=plsc.VectorSubcoreMesh(**_MESH),
        scratch_types=_AGG_SCRATCH,
    )(body)


def _agg_body(g_hbm, src_hbm, dst_hbm, out_hbm, srcc0, srcc1, dstc0, dstc1,
              rows0, rows1, acc_sh, ss0, ss1, sd0, sd1, sg0, sg1):
    cid = lax.axis_index("c")
    sid = lax.axis_index("s")
    wid = cid * NS + sid
    srcc = (srcc0, srcc1)
    dstc = (dstc0, dstc1)
    rows = (rows0, rows1)
    sems = (ss0, ss1)
    semd = (sd0, sd1)
    semg = (sg0, sg1)

    _zero_acc(rows0, acc_sh, sid)
    plsc.subcore_barrier()

    def fetch_s(j, par):
        return pltpu.async_copy(src_hbm.at[wid, j], srcc[par], sems[par])

    def fetch_d(j, par):
        return pltpu.async_copy(dst_hbm.at[wid, j], dstc[par], semd[par])

    def gather(par):
        return pltpu.async_copy(g_hbm.at[srcc[par]], rows[par], semg[par])

    @pl.loop(0, _GROUPS)
    def _(p):
        j0 = p * _GLEN
        fs = {0: fetch_s(j0, 0), 1: fetch_s(j0 + 1, 1)}
        fd = {0: fetch_d(j0, 0), 1: fetch_d(j0 + 1, 1)}
        fs.pop(0).wait()
        g = {0: gather(0)}
        for t in range(_GLEN):
            par = t % 2
            g.pop(t).wait()
            if t + 1 < _GLEN:
                fs.pop(t + 1).wait()
                g[t + 1] = gather(1 - par)
            fd.pop(t).wait()
            pltpu.sync_copy(rows[par], acc_sh.at[dstc[par]], add=True)
            if t + 2 < _GLEN:
                fs[t + 2] = fetch_s(j0 + t + 2, par)
                fd[t + 2] = fetch_d(j0 + t + 2, par)

    plsc.subcore_barrier()
    _copy_out(acc_sh, rows0, rows1, out_hbm, cid, sid)


def _deg_body(g_hbm, src_hbm, dst_hbm, out_hbm, srcc0, srcc1, dstc0, dstc1,
              rows0, rows1, acc_sh, ss0, ss1, sd0, sd1, sg0, sg1):
    cid = lax.axis_index("c")
    sid = lax.axis_index("s")
    wid = cid * NS + sid
    dstc = (dstc0, dstc1)
    semd = (sd0, sd1)

    _zero_acc(rows0, acc_sh, sid)

    one = jnp.ones((16,), jnp.float32)

    @pl.loop(0, ECH)
    def _(r):
        for c in range(CH // 16):
            rows1[r, pl.ds(c * 16, 16)] = one

    plsc.subcore_barrier()

    def fetch_d(j, par):
        return pltpu.async_copy(dst_hbm.at[wid, j], dstc[par], semd[par])

    @pl.loop(0, _GROUPS)
    def _(p):
        j0 = p * _GLEN
        fd = {0: fetch_d(j0, 0), 1: fetch_d(j0 + 1, 1)}
        for t in range(_GLEN):
            par = t % 2
            fd.pop(t).wait()
            pltpu.sync_copy(rows1, acc_sh.at[dstc[par]], add=True)
            if t + 2 < _GLEN:
                fd[t + 2] = fetch_d(j0 + t + 2, par)

    plsc.subcore_barrier()
    _copy_out(acc_sh, rows0, rows1, out_hbm, cid, sid)


_BLK = 1000
_GRID = N_NODES // _BLK


def _prep_body(x_ref, w_ref, degp_ref, g_ref, dis_ref):
    deg = degp_ref[0, :, 0:1] + degp_ref[1, :, 0:1] + 1.0
    d = lax.rsqrt(deg)
    dis_ref[...] = d
    g_ref[...] = d * jnp.dot(x_ref[...], w_ref[...],
                             preferred_element_type=jnp.float32)


_prep_tc = pl.pallas_call(
    _prep_body,
    grid=(_GRID,),
    in_specs=[
        pl.BlockSpec((_BLK, CH), lambda i: (i, 0)),
        pl.BlockSpec((CH, CH), lambda i: (0, 0)),
        pl.BlockSpec((NC, _BLK, CH), lambda i: (0, i, 0)),
    ],
    out_specs=[
        pl.BlockSpec((_BLK, CH), lambda i: (i, 0)),
        pl.BlockSpec((_BLK, 1), lambda i: (i, 0)),
    ],
    out_shape=[
        jax.ShapeDtypeStruct((N_NODES, CH), jnp.float32),
        jax.ShapeDtypeStruct((N_NODES, 1), jnp.float32),
    ],
)


def _comb_body(p_ref, g_ref, dis_ref, b_ref, w_ref, gn_ref):
    d = dis_ref[...]
    h = d * (p_ref[0] + p_ref[1] + g_ref[...]) + b_ref[...]
    r = jnp.maximum(h, 0.0)
    gn_ref[...] = d * jnp.dot(r, w_ref[...],
                              preferred_element_type=jnp.float32)


_comb_tc = pl.pallas_call(
    _comb_body,
    grid=(_GRID,),
    in_specs=[
        pl.BlockSpec((NC, _BLK, CH), lambda i: (0, i, 0)),
        pl.BlockSpec((_BLK, CH), lambda i: (i, 0)),
        pl.BlockSpec((_BLK, 1), lambda i: (i, 0)),
        pl.BlockSpec((1, CH), lambda i: (0, 0)),
        pl.BlockSpec((CH, CH), lambda i: (0, 0)),
    ],
    out_specs=pl.BlockSpec((_BLK, CH), lambda i: (i, 0)),
    out_shape=jax.ShapeDtypeStruct((N_NODES, CH), jnp.float32),
)


def _final_body(p_ref, g_ref, dis_ref, b_ref, o_ref):
    o_ref[...] = (dis_ref[...] * (p_ref[0] + p_ref[1] + g_ref[...])
                  + b_ref[...])


_final_tc = pl.pallas_call(
    _final_body,
    grid=(_GRID,),
    in_specs=[
        pl.BlockSpec((NC, _BLK, CH), lambda i: (0, i, 0)),
        pl.BlockSpec((_BLK, CH), lambda i: (i, 0)),
        pl.BlockSpec((_BLK, 1), lambda i: (i, 0)),
        pl.BlockSpec((1, CH), lambda i: (0, 0)),
    ],
    out_specs=pl.BlockSpec((_BLK, CH), lambda i: (i, 0)),
    out_shape=jax.ShapeDtypeStruct((N_NODES, CH), jnp.float32),
)


@jax.jit
def kernel(x, edge_index, W1, b1, W2, b2, W3, b3):
    pad = NW * CPW * ECH - N_EDGES
    pad_src = (jnp.arange(pad, dtype=jnp.int32) * 79) % N_NODES
    pad_dst = N_NODES + (jnp.arange(pad, dtype=jnp.int32) % (N_PAD - N_NODES))
    look_src = (jnp.arange(NW * 2 * ECH, dtype=jnp.int32) * 83) % N_NODES
    src = jnp.concatenate([edge_index[0], pad_src]).reshape(NW, CPW, ECH)
    dst = jnp.concatenate([edge_index[1], pad_dst]).reshape(NW, CPW, ECH)
    src = jnp.concatenate(
        [src, look_src.reshape(NW, 2, ECH)], axis=1)
    dst = jnp.concatenate(
        [dst, jnp.full((NW, 2, ECH), N_NODES, jnp.int32)], axis=1)

    agg_sc = _make_agg_sc(True)
    deg_sc = _make_agg_sc(False)
    degp = deg_sc(x, src, dst)
    g1, dis = _prep_tc(x, W1, degp)
    p1 = agg_sc(g1, src, dst)
    g2 = _comb_tc(p1, g1, dis, b1.reshape(1, CH), W2)
    p2 = agg_sc(g2, src, dst)
    g3 = _comb_tc(p2, g2, dis, b2.reshape(1, CH), W3)
    p3 = agg_sc(g3, src, dst)
    return _final_tc(p3, g3, dis, b3.reshape(1, CH))


# speedup vs baseline: 24.1069x; 1.0759x over previous
import functools

import jax
import jax.numpy as jnp
from jax import lax
from jax.experimental import pallas as pl
from jax.experimental.pallas import tpu as pltpu
from jax.experimental.pallas import tpu_sc as plsc

N_NODES = 10000
N_EDGES = 320000
CH = 128
ECH = 128
NC = 2
NS = 16
NW = NC * NS
CPW = 80
CPW_ALLOC = CPW + 2
N_PAD = 10112
RPT = N_PAD // NS
_GROUPS = 4
_GLEN = CPW // _GROUPS

_MESH = dict(core_axis_name="c", subcore_axis_name="s", num_cores=NC,
             num_subcores=NS)

_AGG_SCRATCH = [
    pltpu.VMEM((2, ECH), jnp.int32),
    pltpu.VMEM((2, ECH), jnp.int32),
    pltpu.VMEM((2, ECH), jnp.int32),
    pltpu.VMEM((ECH, CH), jnp.float32),
    pltpu.VMEM((ECH, CH), jnp.float32),
    pltpu.VMEM((ECH, CH), jnp.float32),
    pltpu.VMEM_SHARED((N_PAD, CH), jnp.float32),
    pltpu.SemaphoreType.DMA,
    pltpu.SemaphoreType.DMA,
    pltpu.SemaphoreType.DMA,
    pltpu.SemaphoreType.DMA,
    pltpu.SemaphoreType.DMA,
    pltpu.SemaphoreType.DMA,
]


def _zero_fill(buf, rows, width):
    z = jnp.zeros((16,), jnp.float32)

    @pl.loop(0, rows)
    def _(r):
        for c in range(width // 16):
            buf[r, pl.ds(c * 16, 16)] = z


def _zero_acc(zbuf, acc_sh, sid):
    _zero_fill(zbuf, ECH, CH)
    for k in range(RPT // ECH):
        pltpu.sync_copy(zbuf, acc_sh.at[pl.ds(sid * RPT + k * ECH, ECH)])
    rem = RPT % ECH
    if rem:
        pltpu.sync_copy(zbuf.at[pl.ds(0, rem)],
                        acc_sh.at[pl.ds(sid * RPT + (RPT // ECH) * ECH, rem)])


def _copy_out(acc_sh, rows0, rows1, out_hbm, cid, sid):
    base = sid * RPT
    rem = RPT % ECH
    bufs = (rows0, rows1)
    for k in range(RPT // ECH):
        b = bufs[k % 2]
        pltpu.sync_copy(acc_sh.at[pl.ds(base + k * ECH, ECH)], b)
        pltpu.sync_copy(b, out_hbm.at[cid, pl.ds(base + k * ECH, ECH)])
    if rem:
        b2 = base + (RPT // ECH) * ECH
        pltpu.sync_copy(acc_sh.at[pl.ds(b2, rem)], rows0.at[pl.ds(0, rem)])
        pltpu.sync_copy(rows0.at[pl.ds(0, rem)],
                        out_hbm.at[cid, pl.ds(b2, rem)])


@functools.cache
def _make_agg_sc(with_gather: bool):
    body = _agg_body if with_gather else _deg_body
    return functools.partial(
        pl.kernel,
        out_type=jax.ShapeDtypeStruct((NC, N_PAD, CH), jnp.float32),
        mesh=plsc.VectorSubcoreMesh(**_MESH),
        scratch_types=_AGG_SCRATCH,
    )(body)


def _agg_body(g_hbm, idx_hbm, out_hbm, idx0, idx1, idx2,
              rows0, rows1, rows2, acc_sh, si0, si1, si2, sg0, sg1, sg2):
    cid = lax.axis_index("c")
    sid = lax.axis_index("s")
    wid = cid * NS + sid
    idxb = (idx0, idx1, idx2)
    rows = (rows0, rows1, rows2)
    semi = (si0, si1, si2)
    semg = (sg0, sg1, sg2)

    _zero_acc(rows0, acc_sh, sid)
    plsc.subcore_barrier()

    def fetch(j, r):
        return pltpu.async_copy(idx_hbm.at[wid, j], idxb[r], semi[r])

    def gather(r):
        return pltpu.async_copy(g_hbm.at[idxb[r].at[0]], rows[r], semg[r])

    @pl.loop(0, _GROUPS)
    def _(p):
        j0 = p * _GLEN
        f = {i: fetch(j0 + i, i) for i in range(3)}
        f.pop(0).wait()
        g = {0: gather(0)}
        f.pop(1).wait()
        g[1] = gather(1)
        for t in range(_GLEN):
            r = t % 3
            g.pop(t).wait()
            if t + 2 < _GLEN:
                f.pop(t + 2).wait()
                g[t + 2] = gather((t + 2) % 3)
            pltpu.sync_copy(rows[r], acc_sh.at[idxb[r].at[1]], add=True)
            if t + 3 < _GLEN:
                f[t + 3] = fetch(j0 + t + 3, r)

    plsc.subcore_barrier()
    _copy_out(acc_sh, rows0, rows1, out_hbm, cid, sid)


def _deg_body(g_hbm, idx_hbm, out_hbm, idx0, idx1, idx2,
              rows0, rows1, rows2, acc_sh, si0, si1, si2, sg0, sg1, sg2):
    cid = lax.axis_index("c")
    sid = lax.axis_index("s")
    wid = cid * NS + sid
    idxb = (idx0, idx1, idx2)
    semi = (si0, si1, si2)

    _zero_acc(rows0, acc_sh, sid)

    one = jnp.ones((16,), jnp.float32)

    @pl.loop(0, ECH)
    def _(r):
        for c in range(CH // 16):
            rows1[r, pl.ds(c * 16, 16)] = one

    plsc.subcore_barrier()

    def fetch(j, r):
        return pltpu.async_copy(idx_hbm.at[wid, j], idxb[r], semi[r])

    @pl.loop(0, _GROUPS)
    def _(p):
        j0 = p * _GLEN
        f = {i: fetch(j0 + i, i) for i in range(3)}
        for t in range(_GLEN):
            r = t % 3
            f.pop(t).wait()
            pltpu.sync_copy(rows1, acc_sh.at[idxb[r].at[1]], add=True)
            if t + 3 < _GLEN:
                f[t + 3] = fetch(j0 + t + 3, r)

    plsc.subcore_barrier()
    _copy_out(acc_sh, rows0, rows1, out_hbm, cid, sid)


_BLK = 2000
_GRID = N_NODES // _BLK


def _prep_body(x_ref, w_ref, degp_ref, g_ref, dis_ref):
    deg = degp_ref[0, :, 0:1] + degp_ref[1, :, 0:1] + 1.0
    d = lax.rsqrt(deg)
    dis_ref[...] = d
    g_ref[...] = d * jnp.dot(x_ref[...], w_ref[...],
                             preferred_element_type=jnp.float32)


_prep_tc = pl.pallas_call(
    _prep_body,
    grid=(_GRID,),
    in_specs=[
        pl.BlockSpec((_BLK, CH), lambda i: (i, 0)),
        pl.BlockSpec((CH, CH), lambda i: (0, 0)),
        pl.BlockSpec((NC, _BLK, CH), lambda i: (0, i, 0)),
    ],
    out_specs=[
        pl.BlockSpec((_BLK, CH), lambda i: (i, 0)),
        pl.BlockSpec((_BLK, 1), lambda i: (i, 0)),
    ],
    out_shape=[
        jax.ShapeDtypeStruct((N_NODES, CH), jnp.float32),
        jax.ShapeDtypeStruct((N_NODES, 1), jnp.float32),
    ],
)


def _comb_body(p_ref, g_ref, dis_ref, b_ref, w_ref, gn_ref):
    d = dis_ref[...]
    h = d * (p_ref[0] + p_ref[1] + g_ref[...]) + b_ref[...]
    r = jnp.maximum(h, 0.0)
    gn_ref[...] = d * jnp.dot(r, w_ref[...],
                              preferred_element_type=jnp.float32)


_comb_tc = pl.pallas_call(
    _comb_body,
    grid=(_GRID,),
    in_specs=[
        pl.BlockSpec((NC, _BLK, CH), lambda i: (0, i, 0)),
        pl.BlockSpec((_BLK, CH), lambda i: (i, 0)),
        pl.BlockSpec((_BLK, 1), lambda i: (i, 0)),
        pl.BlockSpec((1, CH), lambda i: (0, 0)),
        pl.BlockSpec((CH, CH), lambda i: (0, 0)),
    ],
    out_specs=pl.BlockSpec((_BLK, CH), lambda i: (i, 0)),
    out_shape=jax.ShapeDtypeStruct((N_NODES, CH), jnp.float32),
)


def _final_body(p_ref, g_ref, dis_ref, b_ref, o_ref):
    o_ref[...] = (dis_ref[...] * (p_ref[0] + p_ref[1] + g_ref[...])
                  + b_ref[...])


_final_tc = pl.pallas_call(
    _final_body,
    grid=(_GRID,),
    in_specs=[
        pl.BlockSpec((NC, _BLK, CH), lambda i: (0, i, 0)),
        pl.BlockSpec((_BLK, CH), lambda i: (i, 0)),
        pl.BlockSpec((_BLK, 1), lambda i: (i, 0)),
        pl.BlockSpec((1, CH), lambda i: (0, 0)),
    ],
    out_specs=pl.BlockSpec((_BLK, CH), lambda i: (i, 0)),
    out_shape=jax.ShapeDtypeStruct((N_NODES, CH), jnp.float32),
)


@jax.jit
def kernel(x, edge_index, W1, b1, W2, b2, W3, b3):
    pad = NW * CPW * ECH - N_EDGES
    pad_src = (jnp.arange(pad, dtype=jnp.int32) * 79) % N_NODES
    pad_dst = N_NODES + (jnp.arange(pad, dtype=jnp.int32) % (N_PAD - N_NODES))
    src = jnp.concatenate([edge_index[0], pad_src]).reshape(NW, CPW, 1, ECH)
    dst = jnp.concatenate([edge_index[1], pad_dst]).reshape(NW, CPW, 1, ECH)
    idx = jnp.concatenate([src, dst], axis=2)

    agg_sc = _make_agg_sc(True)
    deg_sc = _make_agg_sc(False)
    degp = deg_sc(x, idx)
    g1, dis = _prep_tc(x, W1, degp)
    p1 = agg_sc(g1, idx)
    g2 = _comb_tc(p1, g1, dis, b1.reshape(1, CH), W2)
    p2 = agg_sc(g2, idx)
    g3 = _comb_tc(p2, g2, dis, b2.reshape(1, CH), W3)
    p3 = agg_sc(g3, idx)
    return _final_tc(p3, g3, dis, b3.reshape(1, CH))


# speedup vs baseline: 24.4437x; 1.0140x over previous
import functools

import jax
import jax.numpy as jnp
from jax import lax
from jax.experimental import pallas as pl
from jax.experimental.pallas import tpu as pltpu
from jax.experimental.pallas import tpu_sc as plsc

N_NODES = 10000
N_EDGES = 320000
CH = 128
ECH = 128
NC = 2
NS = 16
NW = NC * NS
CPW = 80
CPW_ALLOC = CPW + 2
N_PAD = 10112
RPT = N_PAD // NS
_GROUPS = 4
_GLEN = CPW // _GROUPS

_MESH = dict(core_axis_name="c", subcore_axis_name="s", num_cores=NC,
             num_subcores=NS)

_AGG_SCRATCH = [
    pltpu.VMEM((2, ECH), jnp.int32),
    pltpu.VMEM((2, ECH), jnp.int32),
    pltpu.VMEM((2, ECH), jnp.int32),
    pltpu.VMEM((ECH, CH), jnp.float32),
    pltpu.VMEM((ECH, CH), jnp.float32),
    pltpu.VMEM((ECH, CH), jnp.float32),
    pltpu.VMEM_SHARED((N_PAD, CH), jnp.float32),
    pltpu.SemaphoreType.DMA,
    pltpu.SemaphoreType.DMA,
    pltpu.SemaphoreType.DMA,
    pltpu.SemaphoreType.DMA,
    pltpu.SemaphoreType.DMA,
    pltpu.SemaphoreType.DMA,
]


def _zero_fill(buf, rows, width):
    z = jnp.zeros((16,), jnp.float32)

    @pl.loop(0, rows)
    def _(r):
        for c in range(width // 16):
            buf[r, pl.ds(c * 16, 16)] = z


def _slices(base):
    out = []
    for k in range(RPT // ECH):
        out.append((base + k * ECH, ECH))
    if RPT % ECH:
        out.append((base + (RPT // ECH) * ECH, RPT % ECH))
    return out


def _zero_acc(zbuf, acc_sh, sid, sems):
    _zero_fill(zbuf, ECH, CH)
    cps = []
    for i, (off, sz) in enumerate(_slices(sid * RPT)):
        cps.append(pltpu.async_copy(zbuf.at[pl.ds(0, sz)],
                                    acc_sh.at[pl.ds(off, sz)],
                                    sems[i % len(sems)]))
    for c in cps:
        c.wait()


def _copy_out(acc_sh, rows, out_hbm, cid, sid, semi, semg):
    sls = _slices(sid * RPT)
    nk = len(sls)

    def load(k, r):
        off, sz = sls[k]
        return pltpu.async_copy(acc_sh.at[pl.ds(off, sz)],
                                rows[r].at[pl.ds(0, sz)], semi[r])

    def store(k, r):
        off, sz = sls[k]
        return pltpu.async_copy(rows[r].at[pl.ds(0, sz)],
                                out_hbm.at[cid, pl.ds(off, sz)], semg[r])

    ins = {0: load(0, 0)}
    if nk > 1:
        ins[1] = load(1, 1)
    outs = {}
    for k in range(nk):
        r = k % 3
        ins.pop(k).wait()
        outs[k] = store(k, r)
        if k + 2 < nk:
            if k >= 1:
                outs.pop(k - 1).wait()
            ins[k + 2] = load(k + 2, (k + 2) % 3)
    for c in outs.values():
        c.wait()


@functools.cache
def _make_agg_sc(with_gather: bool):
    body = _agg_body if with_gather else _deg_body
    return functools.partial(
        pl.kernel,
        out_type=jax.ShapeDtypeStruct((NC, N_PAD, CH), jnp.float32),
        mesh=plsc.VectorSubcoreMesh(**_MESH),
        scratch_types=_AGG_SCRATCH,
    )(body)


def _agg_body(g_hbm, idx_hbm, out_hbm, idx0, idx1, idx2,
              rows0, rows1, rows2, acc_sh, si0, si1, si2, sg0, sg1, sg2):
    cid = lax.axis_index("c")
    sid = lax.axis_index("s")
    wid = cid * NS + sid
    idxb = (idx0, idx1, idx2)
    rows = (rows0, rows1, rows2)
    semi = (si0, si1, si2)
    semg = (sg0, sg1, sg2)

    _zero_acc(rows0, acc_sh, sid, (si0, si1, si2, sg0, sg1))
    plsc.subcore_barrier()

    def fetch(j, r):
        return pltpu.async_copy(idx_hbm.at[wid, j], idxb[r], semi[r])

    def gather(r):
        return pltpu.async_copy(g_hbm.at[idxb[r].at[0]], rows[r], semg[r])

    @pl.loop(0, _GROUPS)
    def _(p):
        j0 = p * _GLEN
        f = {i: fetch(j0 + i, i) for i in range(3)}
        f.pop(0).wait()
        g = {0: gather(0)}
        f.pop(1).wait()
        g[1] = gather(1)
        for t in range(_GLEN):
            r = t % 3
            g.pop(t).wait()
            if t + 2 < _GLEN:
                f.pop(t + 2).wait()
                g[t + 2] = gather((t + 2) % 3)
            pltpu.sync_copy(rows[r], acc_sh.at[idxb[r].at[1]], add=True)
            if t + 3 < _GLEN:
                f[t + 3] = fetch(j0 + t + 3, r)

    plsc.subcore_barrier()
    _copy_out(acc_sh, rows, out_hbm, cid, sid, semi, semg)


def _deg_body(g_hbm, idx_hbm, out_hbm, idx0, idx1, idx2,
              rows0, rows1, rows2, acc_sh, si0, si1, si2, sg0, sg1, sg2):
    cid = lax.axis_index("c")
    sid = lax.axis_index("s")
    wid = cid * NS + sid
    idxb = (idx0, idx1, idx2)
    rows = (rows0, rows1, rows2)
    semi = (si0, si1, si2)
    semg = (sg0, sg1, sg2)

    _zero_acc(rows0, acc_sh, sid, (si0, si1, si2, sg0, sg1))

    one = jnp.ones((16,), jnp.float32)

    @pl.loop(0, ECH)
    def _(r):
        for c in range(CH // 16):
            rows1[r, pl.ds(c * 16, 16)] = one

    plsc.subcore_barrier()

    def fetch(j, r):
        return pltpu.async_copy(idx_hbm.at[wid, j], idxb[r], semi[r])

    @pl.loop(0, _GROUPS)
    def _(p):
        j0 = p * _GLEN
        f = {i: fetch(j0 + i, i) for i in range(3)}
        for t in range(_GLEN):
            r = t % 3
            f.pop(t).wait()
            pltpu.sync_copy(rows1, acc_sh.at[idxb[r].at[1]], add=True)
            if t + 3 < _GLEN:
                f[t + 3] = fetch(j0 + t + 3, r)

    plsc.subcore_barrier()
    _copy_out(acc_sh, rows, out_hbm, cid, sid, semi, semg)


_BLK = 2000
_GRID = N_NODES // _BLK


def _prep_body(x_ref, w_ref, degp_ref, g_ref, dis_ref):
    deg = degp_ref[0, :, 0:1] + degp_ref[1, :, 0:1] + 1.0
    d = lax.rsqrt(deg)
    dis_ref[...] = d
    g_ref[...] = d * jnp.dot(x_ref[...], w_ref[...],
                             preferred_element_type=jnp.float32)


_prep_tc = pl.pallas_call(
    _prep_body,
    grid=(_GRID,),
    in_specs=[
        pl.BlockSpec((_BLK, CH), lambda i: (i, 0)),
        pl.BlockSpec((CH, CH), lambda i: (0, 0)),
        pl.BlockSpec((NC, _BLK, CH), lambda i: (0, i, 0)),
    ],
    out_specs=[
        pl.BlockSpec((_BLK, CH), lambda i: (i, 0)),
        pl.BlockSpec((_BLK, 1), lambda i: (i, 0)),
    ],
    out_shape=[
        jax.ShapeDtypeStruct((N_NODES, CH), jnp.float32),
        jax.ShapeDtypeStruct((N_NODES, 1), jnp.float32),
    ],
)


def _comb_body(p_ref, g_ref, dis_ref, b_ref, w_ref, gn_ref):
    d = dis_ref[...]
    h = d * (p_ref[0] + p_ref[1] + g_ref[...]) + b_ref[...]
    r = jnp.maximum(h, 0.0)
    gn_ref[...] = d * jnp.dot(r, w_ref[...],
                              preferred_element_type=jnp.float32)


_comb_tc = pl.pallas_call(
    _comb_body,
    grid=(_GRID,),
    in_specs=[
        pl.BlockSpec((NC, _BLK, CH), lambda i: (0, i, 0)),
        pl.BlockSpec((_BLK, CH), lambda i: (i, 0)),
        pl.BlockSpec((_BLK, 1), lambda i: (i, 0)),
        pl.BlockSpec((1, CH), lambda i: (0, 0)),
        pl.BlockSpec((CH, CH), lambda i: (0, 0)),
    ],
    out_specs=pl.BlockSpec((_BLK, CH), lambda i: (i, 0)),
    out_shape=jax.ShapeDtypeStruct((N_NODES, CH), jnp.float32),
)


def _final_body(p_ref, g_ref, dis_ref, b_ref, o_ref):
    o_ref[...] = (dis_ref[...] * (p_ref[0] + p_ref[1] + g_ref[...])
                  + b_ref[...])


_final_tc = pl.pallas_call(
    _final_body,
    grid=(_GRID,),
    in_specs=[
        pl.BlockSpec((NC, _BLK, CH), lambda i: (0, i, 0)),
        pl.BlockSpec((_BLK, CH), lambda i: (i, 0)),
        pl.BlockSpec((_BLK, 1), lambda i: (i, 0)),
        pl.BlockSpec((1, CH), lambda i: (0, 0)),
    ],
    out_specs=pl.BlockSpec((_BLK, CH), lambda i: (i, 0)),
    out_shape=jax.ShapeDtypeStruct((N_NODES, CH), jnp.float32),
)


@jax.jit
def kernel(x, edge_index, W1, b1, W2, b2, W3, b3):
    pad = NW * CPW * ECH - N_EDGES
    pad_src = (jnp.arange(pad, dtype=jnp.int32) * 79) % N_NODES
    pad_dst = N_NODES + (jnp.arange(pad, dtype=jnp.int32) % (N_PAD - N_NODES))
    src = jnp.concatenate([edge_index[0], pad_src]).reshape(NW, CPW, 1, ECH)
    dst = jnp.concatenate([edge_index[1], pad_dst]).reshape(NW, CPW, 1, ECH)
    idx = jnp.concatenate([src, dst], axis=2)

    agg_sc = _make_agg_sc(True)
    deg_sc = _make_agg_sc(False)
    degp = deg_sc(x, idx)
    g1, dis = _prep_tc(x, W1, degp)
    p1 = agg_sc(g1, idx)
    g2 = _comb_tc(p1, g1, dis, b1.reshape(1, CH), W2)
    p2 = agg_sc(g2, idx)
    g3 = _comb_tc(p2, g2, dis, b2.reshape(1, CH), W3)
    p3 = agg_sc(g3, idx)
    return _final_tc(p3, g3, dis, b3.reshape(1, CH))


# speedup vs baseline: 24.9458x; 1.0205x over previous
import functools

import jax
import jax.numpy as jnp
from jax import lax
from jax.experimental import pallas as pl
from jax.experimental.pallas import tpu as pltpu
from jax.experimental.pallas import tpu_sc as plsc

N_NODES = 10000
N_EDGES = 320000
CH = 128
ECH = 128
NC = 2
NS = 16
NW = NC * NS
CPW = 80
CPW_ALLOC = CPW + 2
N_PAD = 10112
RPT = N_PAD // NS
_GROUPS = 2
_GLEN = CPW // _GROUPS

_MESH = dict(core_axis_name="c", subcore_axis_name="s", num_cores=NC,
             num_subcores=NS)

_AGG_SCRATCH = [
    pltpu.VMEM((2, ECH), jnp.int32),
    pltpu.VMEM((2, ECH), jnp.int32),
    pltpu.VMEM((2, ECH), jnp.int32),
    pltpu.VMEM((ECH, CH), jnp.float32),
    pltpu.VMEM((ECH, CH), jnp.float32),
    pltpu.VMEM((ECH, CH), jnp.float32),
    pltpu.VMEM_SHARED((N_PAD, CH), jnp.float32),
    pltpu.SemaphoreType.DMA,
    pltpu.SemaphoreType.DMA,
    pltpu.SemaphoreType.DMA,
    pltpu.SemaphoreType.DMA,
    pltpu.SemaphoreType.DMA,
    pltpu.SemaphoreType.DMA,
]


def _zero_fill(buf, rows, width):
    z = jnp.zeros((16,), jnp.float32)

    @pl.loop(0, rows)
    def _(r):
        for c in range(width // 16):
            buf[r, pl.ds(c * 16, 16)] = z


def _slices(base):
    out = []
    for k in range(RPT // ECH):
        out.append((base + k * ECH, ECH))
    if RPT % ECH:
        out.append((base + (RPT // ECH) * ECH, RPT % ECH))
    return out


def _zero_acc(zbuf, acc_sh, sid, sems):
    _zero_fill(zbuf, ECH, CH)
    cps = []
    for i, (off, sz) in enumerate(_slices(sid * RPT)):
        cps.append(pltpu.async_copy(zbuf.at[pl.ds(0, sz)],
                                    acc_sh.at[pl.ds(off, sz)],
                                    sems[i % len(sems)]))
    for c in cps:
        c.wait()


def _copy_out(acc_sh, rows, out_hbm, cid, sid, semi, semg):
    sls = _slices(sid * RPT)
    nk = len(sls)

    def load(k, r):
        off, sz = sls[k]
        return pltpu.async_copy(acc_sh.at[pl.ds(off, sz)],
                                rows[r].at[pl.ds(0, sz)], semi[r])

    def store(k, r):
        off, sz = sls[k]
        return pltpu.async_copy(rows[r].at[pl.ds(0, sz)],
                                out_hbm.at[cid, pl.ds(off, sz)], semg[r])

    ins = {0: load(0, 0)}
    if nk > 1:
        ins[1] = load(1, 1)
    outs = {}
    for k in range(nk):
        r = k % 3
        ins.pop(k).wait()
        outs[k] = store(k, r)
        if k + 2 < nk:
            if k >= 1:
                outs.pop(k - 1).wait()
            ins[k + 2] = load(k + 2, (k + 2) % 3)
    for c in outs.values():
        c.wait()


@functools.cache
def _make_agg_sc(with_gather: bool):
    body = _agg_body if with_gather else _deg_body
    return functools.partial(
        pl.kernel,
        out_type=jax.ShapeDtypeStruct((NC, N_PAD, CH), jnp.float32),
        mesh=plsc.VectorSubcoreMesh(**_MESH),
        scratch_types=_AGG_SCRATCH,
    )(body)


def _agg_body(g_hbm, idx_hbm, out_hbm, idx0, idx1, idx2,
              rows0, rows1, rows2, acc_sh, si0, si1, si2, sg0, sg1, sg2):
    cid = lax.axis_index("c")
    sid = lax.axis_index("s")
    wid = cid * NS + sid
    idxb = (idx0, idx1, idx2)
    rows = (rows0, rows1, rows2)
    semi = (si0, si1, si2)
    semg = (sg0, sg1, sg2)

    _zero_acc(rows0, acc_sh, sid, (si0, si1, si2, sg0, sg1))
    plsc.subcore_barrier()

    def fetch(j, r):
        return pltpu.async_copy(idx_hbm.at[wid, j], idxb[r], semi[r])

    def gather(r):
        return pltpu.async_copy(g_hbm.at[idxb[r].at[0]], rows[r], semg[r])

    @pl.loop(0, _GROUPS)
    def _(p):
        j0 = p * _GLEN
        f = {i: fetch(j0 + i, i) for i in range(3)}
        f.pop(0).wait()
        g = {0: gather(0)}
        f.pop(1).wait()
        g[1] = gather(1)
        for t in range(_GLEN):
            r = t % 3
            g.pop(t).wait()
            if t + 2 < _GLEN:
                f.pop(t + 2).wait()
                g[t + 2] = gather((t + 2) % 3)
            pltpu.sync_copy(rows[r], acc_sh.at[idxb[r].at[1]], add=True)
            if t + 3 < _GLEN:
                f[t + 3] = fetch(j0 + t + 3, r)

    plsc.subcore_barrier()
    _copy_out(acc_sh, rows, out_hbm, cid, sid, semi, semg)


def _deg_body(g_hbm, idx_hbm, out_hbm, idx0, idx1, idx2,
              rows0, rows1, rows2, acc_sh, si0, si1, si2, sg0, sg1, sg2):
    cid = lax.axis_index("c")
    sid = lax.axis_index("s")
    wid = cid * NS + sid
    idxb = (idx0, idx1, idx2)
    rows = (rows0, rows1, rows2)
    semi = (si0, si1, si2)
    semg = (sg0, sg1, sg2)

    _zero_acc(rows0, acc_sh, sid, (si0, si1, si2, sg0, sg1))

    one = jnp.ones((16,), jnp.float32)

    @pl.loop(0, ECH)
    def _(r):
        for c in range(CH // 16):
            rows1[r, pl.ds(c * 16, 16)] = one

    plsc.subcore_barrier()

    def fetch(j, r):
        return pltpu.async_copy(idx_hbm.at[wid, j], idxb[r], semi[r])

    @pl.loop(0, _GROUPS)
    def _(p):
        j0 = p * _GLEN
        f = {i: fetch(j0 + i, i) for i in range(3)}
        for t in range(_GLEN):
            r = t % 3
            f.pop(t).wait()
            pltpu.sync_copy(rows1, acc_sh.at[idxb[r].at[1]], add=True)
            if t + 3 < _GLEN:
                f[t + 3] = fetch(j0 + t + 3, r)

    plsc.subcore_barrier()
    _copy_out(acc_sh, rows, out_hbm, cid, sid, semi, semg)


_BLK = 5000
_GRID = N_NODES // _BLK


def _prep_body(x_ref, w_ref, degp_ref, g_ref, dis_ref):
    deg = degp_ref[0, :, 0:1] + degp_ref[1, :, 0:1] + 1.0
    d = lax.rsqrt(deg)
    dis_ref[...] = d
    g_ref[...] = d * jnp.dot(x_ref[...], w_ref[...],
                             preferred_element_type=jnp.float32)


_prep_tc = pl.pallas_call(
    _prep_body,
    grid=(_GRID,),
    in_specs=[
        pl.BlockSpec((_BLK, CH), lambda i: (i, 0)),
        pl.BlockSpec((CH, CH), lambda i: (0, 0)),
        pl.BlockSpec((NC, _BLK, CH), lambda i: (0, i, 0)),
    ],
    out_specs=[
        pl.BlockSpec((_BLK, CH), lambda i: (i, 0)),
        pl.BlockSpec((_BLK, 1), lambda i: (i, 0)),
    ],
    out_shape=[
        jax.ShapeDtypeStruct((N_NODES, CH), jnp.float32),
        jax.ShapeDtypeStruct((N_NODES, 1), jnp.float32),
    ],
)


def _comb_body(p_ref, g_ref, dis_ref, b_ref, w_ref, gn_ref):
    d = dis_ref[...]
    h = d * (p_ref[0] + p_ref[1] + g_ref[...]) + b_ref[...]
    r = jnp.maximum(h, 0.0)
    gn_ref[...] = d * jnp.dot(r, w_ref[...],
                              preferred_element_type=jnp.float32)


_comb_tc = pl.pallas_call(
    _comb_body,
    grid=(_GRID,),
    in_specs=[
        pl.BlockSpec((NC, _BLK, CH), lambda i: (0, i, 0)),
        pl.BlockSpec((_BLK, CH), lambda i: (i, 0)),
        pl.BlockSpec((_BLK, 1), lambda i: (i, 0)),
        pl.BlockSpec((1, CH), lambda i: (0, 0)),
        pl.BlockSpec((CH, CH), lambda i: (0, 0)),
    ],
    out_specs=pl.BlockSpec((_BLK, CH), lambda i: (i, 0)),
    out_shape=jax.ShapeDtypeStruct((N_NODES, CH), jnp.float32),
)


def _final_body(p_ref, g_ref, dis_ref, b_ref, o_ref):
    o_ref[...] = (dis_ref[...] * (p_ref[0] + p_ref[1] + g_ref[...])
                  + b_ref[...])


_final_tc = pl.pallas_call(
    _final_body,
    grid=(_GRID,),
    in_specs=[
        pl.BlockSpec((NC, _BLK, CH), lambda i: (0, i, 0)),
        pl.BlockSpec((_BLK, CH), lambda i: (i, 0)),
        pl.BlockSpec((_BLK, 1), lambda i: (i, 0)),
        pl.BlockSpec((1, CH), lambda i: (0, 0)),
    ],
    out_specs=pl.BlockSpec((_BLK, CH), lambda i: (i, 0)),
    out_shape=jax.ShapeDtypeStruct((N_NODES, CH), jnp.float32),
)


@jax.jit
def kernel(x, edge_index, W1, b1, W2, b2, W3, b3):
    pad = NW * CPW * ECH - N_EDGES
    pad_src = (jnp.arange(pad, dtype=jnp.int32) * 79) % N_NODES
    pad_dst = N_NODES + (jnp.arange(pad, dtype=jnp.int32) % (N_PAD - N_NODES))
    src = jnp.concatenate([edge_index[0], pad_src]).reshape(NW, CPW, 1, ECH)
    dst = jnp.concatenate([edge_index[1], pad_dst]).reshape(NW, CPW, 1, ECH)
    idx = jnp.concatenate([src, dst], axis=2)

    agg_sc = _make_agg_sc(True)
    deg_sc = _make_agg_sc(False)
    degp = deg_sc(x, idx)
    g1, dis = _prep_tc(x, W1, degp)
    p1 = agg_sc(g1, idx)
    g2 = _comb_tc(p1, g1, dis, b1.reshape(1, CH), W2)
    p2 = agg_sc(g2, idx)
    g3 = _comb_tc(p2, g2, dis, b2.reshape(1, CH), W3)
    p3 = agg_sc(g3, idx)
    return _final_tc(p3, g3, dis, b3.reshape(1, CH))
